# baseline (speedup 1.0000x reference)
import functools

import jax
import jax.numpy as jnp
from jax import lax
from jax.experimental import pallas as pl
from jax.experimental.pallas import tpu as pltpu
from jax.experimental.pallas import tpu_sc as plsc

B = 16384
L = 200
D = 64
N = B * L

VY, VM, VW, VH = 100, 13, 7, 24
MWH = VM * VW * VH
WH = VW * VH
NCOMB = VY * MWH

NC = 2
NS = 16
NW = NC * NS

NB_B = 2
K = NB_B * L
CH = 112
OV = L - CH
PER_B = B // NW
NBLK = PER_B // NB_B


def _sc_body(year_h, month_h, wday_h, hour_h, yt_h, mt_h, wt_h, ht_h, out_h,
             yt_v, mt_v, wt_v, ht_v, stage_v,
             iy_v, im_v, iw_v, ih_v, fidx_v, rows_v, comb_s,
             sem_b, sem_i, sem_g, sem_o):
    cid = lax.axis_index("c")
    sid = lax.axis_index("s")
    wid = sid * NC + cid

    pltpu.sync_copy(yt_h, yt_v)
    pltpu.sync_copy(mt_h, mt_v)
    pltpu.sync_copy(wt_h, wt_v)
    pltpu.sync_copy(ht_h, ht_v)

    comb_c = comb_s.at[cid]
    ym_lo = sid * VY // NS * VM
    ym_hi = (sid + 1) * VY // NS * VM

    def build_ym(i, _):
        ym = ym_lo + i
        y = ym // VM
        m = ym - y * VM
        p = i % 2
        @pl.when(i >= 2)
        def _():
            pltpu.make_async_copy(stage_v.at[p], comb_c.at[pl.ds(0, WH)],
                                  sem_b).wait()

        ymv = [yt_v[y, pl.ds(q * 16, 16)] + mt_v[m, pl.ds(q * 16, 16)]
               for q in range(D // 16)]

        def build_w(w, _):
            ymw = [ymv[q] + wt_v[w, pl.ds(q * 16, 16)]
                   for q in range(D // 16)]

            def build_h(h, _):
                r = w * VH + h
                for q in range(D // 16):
                    s = pl.ds(q * 16, 16)
                    stage_v[p, r, s] = ymw[q] + ht_v[h, s]
                return 0
            lax.fori_loop(0, VH, build_h, 0, unroll=4)
            return 0
        lax.fori_loop(0, VW, build_w, 0)
        pltpu.async_copy(stage_v.at[p], comb_c.at[pl.ds(ym * WH, WH)], sem_b)
        return 0

    lax.fori_loop(0, ym_hi - ym_lo, build_ym, 0)
    for _ in range(2):
        pltpu.make_async_copy(stage_v.at[0], comb_c.at[pl.ds(0, WH)],
                              sem_b).wait()

    plsc.subcore_barrier()

    base_b = wid * PER_B

    def start_idx(blk, p):
        base = pl.multiple_of((base_b + blk * NB_B) * L, K)
        pltpu.async_copy(year_h.at[pl.ds(base, K)], iy_v.at[p], sem_i)
        pltpu.async_copy(month_h.at[pl.ds(base, K)], im_v.at[p], sem_i)
        pltpu.async_copy(wday_h.at[pl.ds(base, K)], iw_v.at[p], sem_i)
        pltpu.async_copy(hour_h.at[pl.ds(base, K)], ih_v.at[p], sem_i)

    def wait_idx(p):
        for r in (iy_v, im_v, iw_v, ih_v):
            pltpu.make_async_copy(year_h.at[pl.ds(0, K)], r.at[p],
                                  sem_i).wait()

    start_idx(0, 0)

    def block(blk, _):
        p = blk % 2
        b0 = base_b + blk * NB_B
        wait_idx(p)

        @pl.when(blk + 1 < NBLK)
        def _():
            start_idx(blk + 1, 1 - p)

        def chunk(c, _):
            bb = c // 2
            q0 = bb * L + (c % 2) * OV

            def group(gg, _):
                off = q0 + gg * 16
                fused = (iy_v[p, pl.ds(off, 16)] * MWH
                         + im_v[p, pl.ds(off, 16)] * WH
                         + iw_v[p, pl.ds(off, 16)] * VH
                         + ih_v[p, pl.ds(off, 16)])
                fidx_v[p, c, pl.ds(gg * 16, 16)] = fused
                return 0

            lax.fori_loop(0, CH // 16, group, 0, unroll=True)
            return 0

        lax.fori_loop(0, 2 * NB_B, chunk, 0)

        @pl.when(blk >= 2)
        def _():
            pltpu.make_async_copy(rows_v.at[p], out_h.at[pl.ds(0, NB_B)],
                                  sem_o).wait()

        gathers = [
            pltpu.async_copy(
                comb_c.at[fidx_v.at[p, bb * 2 + half]],
                rows_v.at[p, bb].at[pl.ds(half * OV, CH)], sem_g)
            for bb in range(NB_B) for half in range(2)
        ]
        for g in gathers:
            g.wait()

        pltpu.async_copy(rows_v.at[p], out_h.at[pl.ds(b0, NB_B)], sem_o)
        return 0

    lax.fori_loop(0, NBLK, block, 0)

    for _ in range(2):
        pltpu.make_async_copy(rows_v.at[0], out_h.at[pl.ds(0, NB_B)],
                              sem_o).wait()


@jax.jit
def _run(year, month, weekday, hour, yt, mt, wt, ht):
    f = pl.kernel(
        _sc_body,
        out_type=jax.ShapeDtypeStruct((B, L, D), jnp.float32),
        mesh=plsc.VectorSubcoreMesh(core_axis_name="c", subcore_axis_name="s"),
        compiler_params=pltpu.CompilerParams(needs_layout_passes=False,
                                             use_tc_tiling_on_sc=False),
        scratch_types=[
            pltpu.VMEM((VY, D), jnp.float32),
            pltpu.VMEM((VM, D), jnp.float32),
            pltpu.VMEM((VW, D), jnp.float32),
            pltpu.VMEM((VH, D), jnp.float32),
            pltpu.VMEM((2, WH, D), jnp.float32),
            pltpu.VMEM((2, K), jnp.int32),
            pltpu.VMEM((2, K), jnp.int32),
            pltpu.VMEM((2, K), jnp.int32),
            pltpu.VMEM((2, K), jnp.int32),
            pltpu.VMEM((2, 2 * NB_B, CH), jnp.int32),
            pltpu.VMEM((2, NB_B, L, D), jnp.float32),
            pltpu.HBM((NC, NCOMB, D), jnp.float32),
            pltpu.SemaphoreType.DMA,
            pltpu.SemaphoreType.DMA,
            pltpu.SemaphoreType.DMA,
            pltpu.SemaphoreType.DMA,
        ],
    )
    return f(year, month, weekday, hour, yt, mt, wt, ht)


def kernel(year, month, weekday, hour, year_table, month_table, weekday_table, hour_table):
    return _run(
        year.reshape(-1).astype(jnp.int32),
        month.reshape(-1).astype(jnp.int32),
        weekday.reshape(-1).astype(jnp.int32),
        hour.reshape(-1).astype(jnp.int32),
        year_table, month_table, weekday_table, hour_table,
    )


# speedup vs baseline: 1.1246x; 1.1246x over previous
import functools

import jax
import jax.numpy as jnp
from jax import lax
from jax.experimental import pallas as pl
from jax.experimental.pallas import tpu as pltpu
from jax.experimental.pallas import tpu_sc as plsc

B = 16384
L = 200
D = 64
N = B * L

VY, VM, VW, VH = 100, 13, 7, 24
MWH = VM * VW * VH
WH = VW * VH
NCOMB = VY * MWH

NC = 2
NS = 16
NW = NC * NS

K = 512
CH = 128
NCH = K // CH
PER_W = N // NW
NBLK = PER_W // K


def _sc_body(year_h, month_h, wday_h, hour_h, yt_h, mt_h, wt_h, ht_h, out_h,
             yt_v, mt_v, wt_v, ht_v, stage_v,
             iy_v, im_v, iw_v, ih_v, fidx_v, rows_v, comb_s,
             sem_b, sem_i, sem_g, sem_o):
    cid = lax.axis_index("c")
    sid = lax.axis_index("s")
    wid = sid * NC + cid

    pltpu.sync_copy(yt_h, yt_v)
    pltpu.sync_copy(mt_h, mt_v)
    pltpu.sync_copy(wt_h, wt_v)
    pltpu.sync_copy(ht_h, ht_v)

    comb_c = comb_s.at[cid]
    ym_lo = sid * VY // NS * VM
    ym_hi = (sid + 1) * VY // NS * VM

    def build_ym(i, _):
        ym = ym_lo + i
        y = ym // VM
        m = ym - y * VM
        p = i % 2
        @pl.when(i >= 2)
        def _():
            pltpu.make_async_copy(stage_v.at[p], comb_c.at[pl.ds(0, WH)],
                                  sem_b).wait()

        ymv = [yt_v[y, pl.ds(q * 16, 16)] + mt_v[m, pl.ds(q * 16, 16)]
               for q in range(D // 16)]

        def build_w(w, _):
            ymw = [ymv[q] + wt_v[w, pl.ds(q * 16, 16)]
                   for q in range(D // 16)]

            @plsc.parallel_loop(0, VH, 1, unroll=8)
            def build_h(h):
                r = w * VH + h
                for q in range(D // 16):
                    s = pl.ds(q * 16, 16)
                    stage_v[p, r, s] = ymw[q] + ht_v[h, s]

            return 0
        lax.fori_loop(0, VW, build_w, 0)
        pltpu.async_copy(stage_v.at[p], comb_c.at[pl.ds(ym * WH, WH)], sem_b)
        return 0

    lax.fori_loop(0, ym_hi - ym_lo, build_ym, 0)
    for _ in range(2):
        pltpu.make_async_copy(stage_v.at[0], comb_c.at[pl.ds(0, WH)],
                              sem_b).wait()

    plsc.subcore_barrier()

    base_w = wid * PER_W

    def start_idx(blk, p):
        base = pl.multiple_of(base_w + blk * K, K)
        pltpu.async_copy(year_h.at[pl.ds(base, K)], iy_v.at[p], sem_i)
        pltpu.async_copy(month_h.at[pl.ds(base, K)], im_v.at[p], sem_i)
        pltpu.async_copy(wday_h.at[pl.ds(base, K)], iw_v.at[p], sem_i)
        pltpu.async_copy(hour_h.at[pl.ds(base, K)], ih_v.at[p], sem_i)

    def wait_idx(p):
        for r in (iy_v, im_v, iw_v, ih_v):
            pltpu.make_async_copy(year_h.at[pl.ds(0, K)], r.at[p],
                                  sem_i).wait()

    start_idx(0, 0)

    def block(blk, _):
        p = blk % 2
        base = pl.multiple_of(base_w + blk * K, K)
        wait_idx(p)

        @pl.when(blk + 1 < NBLK)
        def _():
            start_idx(blk + 1, 1 - p)

        for j in range(NCH):
            def group(gg, _):
                off = j * CH + gg * 16
                fused = (iy_v[p, pl.ds(off, 16)] * MWH
                         + im_v[p, pl.ds(off, 16)] * WH
                         + iw_v[p, pl.ds(off, 16)] * VH
                         + ih_v[p, pl.ds(off, 16)])
                fidx_v[p, j, pl.ds(gg * 16, 16)] = fused
                return 0

            lax.fori_loop(0, CH // 16, group, 0, unroll=True)

        @pl.when(blk >= 2)
        def _():
            pltpu.make_async_copy(rows_v.at[p], out_h.at[pl.ds(0, K)],
                                  sem_o).wait()

        gathers = [
            pltpu.async_copy(comb_c.at[fidx_v.at[p, j]],
                             rows_v.at[p].at[pl.ds(j * CH, CH)], sem_g)
            for j in range(NCH)
        ]
        for g in gathers:
            g.wait()

        pltpu.async_copy(rows_v.at[p], out_h.at[pl.ds(base, K)], sem_o)
        return 0

    lax.fori_loop(0, NBLK, block, 0)

    for _ in range(2):
        pltpu.make_async_copy(rows_v.at[0], out_h.at[pl.ds(0, K)],
                              sem_o).wait()


@jax.jit
def _run(year, month, weekday, hour, yt, mt, wt, ht):
    f = pl.kernel(
        _sc_body,
        out_type=jax.ShapeDtypeStruct((N, D), jnp.float32),
        mesh=plsc.VectorSubcoreMesh(core_axis_name="c", subcore_axis_name="s"),
        compiler_params=pltpu.CompilerParams(needs_layout_passes=False,
                                             use_tc_tiling_on_sc=False),
        scratch_types=[
            pltpu.VMEM((VY, D), jnp.float32),
            pltpu.VMEM((VM, D), jnp.float32),
            pltpu.VMEM((VW, D), jnp.float32),
            pltpu.VMEM((VH, D), jnp.float32),
            pltpu.VMEM((2, WH, D), jnp.float32),
            pltpu.VMEM((2, K), jnp.int32),
            pltpu.VMEM((2, K), jnp.int32),
            pltpu.VMEM((2, K), jnp.int32),
            pltpu.VMEM((2, K), jnp.int32),
            pltpu.VMEM((2, NCH, CH), jnp.int32),
            pltpu.VMEM((2, K, D), jnp.float32),
            pltpu.HBM((NC, NCOMB, D), jnp.float32),
            pltpu.SemaphoreType.DMA,
            pltpu.SemaphoreType.DMA,
            pltpu.SemaphoreType.DMA,
            pltpu.SemaphoreType.DMA,
        ],
    )
    return f(year, month, weekday, hour, yt, mt, wt, ht)


def kernel(year, month, weekday, hour, year_table, month_table, weekday_table, hour_table):
    out = _run(
        year.reshape(-1).astype(jnp.int32),
        month.reshape(-1).astype(jnp.int32),
        weekday.reshape(-1).astype(jnp.int32),
        hour.reshape(-1).astype(jnp.int32),
        year_table, month_table, weekday_table, hour_table,
    )
    return out.reshape(B, L, D)
